# Initial kernel scaffold; baseline (speedup 1.0000x reference)
import jax
import jax.numpy as jnp
from jax.experimental import pallas as pl


def kernel(embedding, logit_bias, hidden_states, embedding_bias, temperatures, top_ps, top_ks, min_ps, presence_penalties, frequency_penalties, repetition_penalties, prompt_tokens, output_tokens):
    raise NotImplementedError("write your pallas kernel here")


# speedup vs baseline: 1.6464x; 1.6464x over previous
import functools
import jax
import jax.numpy as jnp
from jax import lax
from jax.experimental import pallas as pl
from jax.experimental.pallas import tpu as pltpu

NEG = -1e10
K = 128
BV = 512
CH = 32
BIGI = 2 ** 30


def _k1_body(nb, n_pchunk, n_tchunk, v_real,
             h_ref, emb_ref, eb_ref, toks_ref, rp_ref, fp_ref, pp_ref, t_ref,
             valsT_ref, idxT_ref, stats_ref, scr_ref):
    j = pl.program_id(0)
    l = lax.dot_general(h_ref[...], emb_ref[...],
                        (((1,), (1,)), ((), ())),
                        preferred_element_type=jnp.float32)
    l = l + eb_ref[...]
    bvi = l.shape[1]
    col = j * bvi + lax.broadcasted_iota(jnp.int32, (1, bvi), 1)
    toks = toks_ref[...]
    b = toks.shape[0]
    pm = jnp.zeros((b, bvi), jnp.float32)
    oc = jnp.zeros((b, bvi), jnp.float32)
    for c in range(n_pchunk + n_tchunk):
        tk = toks[:, c * CH:(c + 1) * CH]
        cmp = (tk[:, :, None] == col[None, :, :]).astype(jnp.float32)
        s = jnp.sum(cmp, axis=1)
        pm = pm + s
        if c >= n_pchunk:
            oc = oc + s
    rp = rp_ref[...]
    l = jnp.where(pm > 0.0, jnp.where(l > 0.0, l / rp, l * rp), l)
    l = l - fp_ref[...] * oc
    l = l - pp_ref[...] * (oc > 0.0).astype(jnp.float32)
    l = l / t_ref[...]
    l = jnp.where(col < v_real, l, NEG)
    scr_ref[j, :, :] = l

    @pl.when(j == nb - 1)
    def _final():
        x = scr_ref[...]
        m_full = jnp.max(jnp.max(x, axis=2), axis=0)
        s_full = jnp.sum(jnp.sum(jnp.exp(x - m_full[None, :, None]),
                                 axis=2), axis=0)
        col3 = (lax.broadcasted_iota(jnp.int32, x.shape, 0) * bvi
                + lax.broadcasted_iota(jnp.int32, x.shape, 2))

        def body(i, carry):
            valsT, idxT = carry
            xx = scr_ref[...]
            m = jnp.max(jnp.max(xx, axis=2), axis=0)
            eq = xx == m[None, :, None]
            idx = jnp.min(jnp.min(jnp.where(eq, col3, BIGI), axis=2),
                          axis=0)
            scr_ref[...] = jnp.where(col3 == idx[None, :, None], NEG, xx)
            rsel = lax.broadcasted_iota(jnp.int32, valsT.shape, 0) == i
            valsT = jnp.where(rsel, m[None, :], valsT)
            idxT = jnp.where(rsel, idx[None, :], idxT)
            return valsT, idxT

        v0 = jnp.zeros((K, b), jnp.float32)
        i0 = jnp.zeros((K, b), jnp.int32)
        valsT, idxT = lax.fori_loop(0, K, body, (v0, i0))
        valsT_ref[...] = valsT
        idxT_ref[...] = idxT
        c8 = lax.broadcasted_iota(jnp.int32, (b, 8), 1)
        stats_ref[...] = jnp.where(c8 == 0, m_full[:, None],
                                   jnp.where(c8 == 1, s_full[:, None], 0.0))


def _k2_body(v_real, valsT_ref, idxT_ref, stats_ref, lb_ref,
             tk_ref, tp_ref, mp_ref,
             probsT_ref, logpT_ref, stats2_ref, nt_ref):
    vals = valsT_ref[...].T
    b = vals.shape[0]
    m_full = stats_ref[:, 0:1]
    s_full = stats_ref[:, 1:2]
    lane = lax.broadcasted_iota(jnp.int32, (b, K), 1)
    k_mask = lane >= tk_ref[...]
    probs_sort = jnp.exp(vals - m_full) / s_full
    tri = (lax.broadcasted_iota(jnp.int32, (K, K), 0)
           <= lax.broadcasted_iota(jnp.int32, (K, K), 1)).astype(jnp.float32)
    cs = lax.dot_general(probs_sort, tri, (((1,), (0,)), ((), ())),
                         preferred_element_type=jnp.float32)
    p_mask = (cs - probs_sort) > tp_ref[...]
    v1 = jnp.where(k_mask | p_mask, NEG, vals)
    m2 = jnp.max(v1, axis=1, keepdims=True)
    e2 = jnp.exp(v1 - m2)
    s2 = jnp.sum(e2, axis=1, keepdims=True)
    probs2 = e2 / s2
    top_prob = jnp.max(probs2, axis=1, keepdims=True)
    v2 = jnp.where(probs2 < mp_ref[...] * top_prob, NEG, v1)
    surv = (v2 > NEG / 2).astype(jnp.float32)
    m3 = jnp.max(v2, axis=1, keepdims=True)
    lse3 = m3 + jnp.log(jnp.sum(jnp.exp(v2 - m3), axis=1, keepdims=True))
    colv = lax.broadcasted_iota(jnp.int32, lb_ref.shape, 1)

    def gbody(i, acc):
        idv = idxT_ref[i, :]
        match = colv == idv[:, None]
        g = jnp.sum(jnp.where(match, lb_ref[...], 0.0), axis=1)
        rsel = lax.broadcasted_iota(jnp.int32, acc.shape, 0) == i
        return jnp.where(rsel, g[None, :], acc)

    gT = lax.fori_loop(0, K, gbody, jnp.zeros((K, b), jnp.float32))
    v3 = v2 - lse3 + gT.T * surv
    m4 = jnp.max(v3, axis=1, keepdims=True)
    e4 = jnp.exp(v3 - m4)
    s4 = jnp.sum(e4, axis=1, keepdims=True)
    probs_v = e4 / s4
    lse4 = m4 + jnp.log(s4)
    logp_v = v3 - lse4
    c_fill = NEG - lse3 - lse4
    am = jnp.argmax(v3, axis=1)
    nt = jnp.sum(jnp.where(lane == am[:, None], idxT_ref[...].T, 0), axis=1)
    probsT_ref[...] = probs_v.T
    logpT_ref[...] = logp_v.T
    c8 = lax.broadcasted_iota(jnp.int32, (b, 8), 1)
    stats2_ref[...] = jnp.where(c8 == 0, c_fill, 0.0)
    nt_ref[...] = jnp.where(c8 == 0, nt[:, None], 0)


def _k3_body(idxT_ref, probsT_ref, logpT_ref, stats2_ref, op_ref, ol_ref):
    j = pl.program_id(0)
    b, bvi = op_ref.shape
    col = j * bvi + lax.broadcasted_iota(jnp.int32, (b, bvi), 1)
    c_fill = stats2_ref[:, 0:1]

    def body(i, carry):
        acc_p, acc_l = carry
        idv = idxT_ref[i, :]
        match = col == idv[:, None]
        pv = probsT_ref[i, :]
        lv = logpT_ref[i, :]
        acc_p = jnp.where(match, pv[:, None], acc_p)
        acc_l = jnp.where(match, lv[:, None], acc_l)
        return acc_p, acc_l

    p0 = jnp.zeros((b, bvi), jnp.float32)
    l0 = jnp.broadcast_to(c_fill, (b, bvi))
    acc_p, acc_l = lax.fori_loop(0, K, body, (p0, l0))
    op_ref[...] = acc_p
    ol_ref[...] = acc_l


def kernel(embedding, logit_bias, hidden_states, embedding_bias, temperatures,
           top_ps, top_ks, min_ps, presence_penalties, frequency_penalties,
           repetition_penalties, prompt_tokens, output_tokens):
    v, d = embedding.shape
    b = hidden_states.shape[0]
    nb = pl.cdiv(v, BV)
    toks = jnp.concatenate([prompt_tokens, output_tokens], axis=1)
    n_pchunk = prompt_tokens.shape[1] // CH
    n_tchunk = output_tokens.shape[1] // CH
    eb2 = embedding_bias.reshape(1, v)
    col = lambda a: a.reshape(b, 1).astype(jnp.float32)

    valsT, idxT, stats = pl.pallas_call(
        functools.partial(_k1_body, nb, n_pchunk, n_tchunk, v),
        grid=(nb,),
        in_specs=[
            pl.BlockSpec((b, d), lambda j: (0, 0)),
            pl.BlockSpec((BV, d), lambda j: (j, 0)),
            pl.BlockSpec((1, BV), lambda j: (0, j)),
            pl.BlockSpec(toks.shape, lambda j: (0, 0)),
            pl.BlockSpec((b, 1), lambda j: (0, 0)),
            pl.BlockSpec((b, 1), lambda j: (0, 0)),
            pl.BlockSpec((b, 1), lambda j: (0, 0)),
            pl.BlockSpec((b, 1), lambda j: (0, 0)),
        ],
        out_specs=[
            pl.BlockSpec((K, b), lambda j: (0, 0)),
            pl.BlockSpec((K, b), lambda j: (0, 0)),
            pl.BlockSpec((b, 8), lambda j: (0, 0)),
        ],
        out_shape=[
            jax.ShapeDtypeStruct((K, b), jnp.float32),
            jax.ShapeDtypeStruct((K, b), jnp.int32),
            jax.ShapeDtypeStruct((b, 8), jnp.float32),
        ],
        scratch_shapes=[pltpu.VMEM((nb, b, BV), jnp.float32)],
    )(hidden_states, embedding, eb2, toks,
      col(repetition_penalties), col(frequency_penalties),
      col(presence_penalties), col(temperatures))

    probsT, logpT, stats2, nt8 = pl.pallas_call(
        functools.partial(_k2_body, v),
        out_shape=[
            jax.ShapeDtypeStruct((K, b), jnp.float32),
            jax.ShapeDtypeStruct((K, b), jnp.float32),
            jax.ShapeDtypeStruct((b, 8), jnp.float32),
            jax.ShapeDtypeStruct((b, 8), jnp.int32),
        ],
    )(valsT, idxT, stats, logit_bias,
      top_ks.reshape(b, 1), top_ps.reshape(b, 1), min_ps.reshape(b, 1))

    probs, logprobs = pl.pallas_call(
        _k3_body,
        grid=(nb,),
        in_specs=[
            pl.BlockSpec((K, b), lambda j: (0, 0)),
            pl.BlockSpec((K, b), lambda j: (0, 0)),
            pl.BlockSpec((K, b), lambda j: (0, 0)),
            pl.BlockSpec((b, 8), lambda j: (0, 0)),
        ],
        out_specs=[
            pl.BlockSpec((b, BV), lambda j: (0, j)),
            pl.BlockSpec((b, BV), lambda j: (0, j)),
        ],
        out_shape=[
            jax.ShapeDtypeStruct((b, v), jnp.float32),
            jax.ShapeDtypeStruct((b, v), jnp.float32),
        ],
    )(idxT, probsT, logpT, stats2)

    next_tokens = nt8[:, 0]
    return (next_tokens, probs, logprobs)
